# Initial kernel scaffold; baseline (speedup 1.0000x reference)
import jax
import jax.numpy as jnp
from jax.experimental import pallas as pl


def kernel(x, edge_index, W_msg, b_msg, W1, b1, W2, b2, W3, b3):
    raise NotImplementedError("write your pallas kernel here")


# speedup vs baseline: 8.3074x; 8.3074x over previous
import functools

import jax
import jax.numpy as jnp
from jax import lax
from jax.experimental import pallas as pl
from jax.experimental.pallas import tpu as pltpu
from jax.experimental.pallas import tpu_sc as plsc

N = 50000
E = 800000
SD = 64
MC = 64
H = 32
HH = 16

NC = 2
NS = 16
BATCH = 128
NB = 392
NSTG = 4
SB = NB // NSTG
EPT = NB * BATCH
EP = NS * EPT
ACC_ROWS = 50176
ZCH = 784
ROWS_PER_TILE = 4 * ZCH


def _sc_segment_sum(y2, src_p, dst_p):
    mesh = plsc.VectorSubcoreMesh(core_axis_name="c", subcore_axis_name="s")

    @functools.partial(
        pl.kernel,
        out_type=jax.ShapeDtypeStruct((NC, N, HH), jnp.float32),
        mesh=mesh,
        compiler_params=pltpu.CompilerParams(use_tc_tiling_on_sc=False),
        scratch_types=[
            pltpu.VMEM((SB, BATCH), jnp.int32),
            pltpu.VMEM((SB, BATCH), jnp.int32),
            pltpu.VMEM((BATCH, HH), jnp.float32),
            pltpu.VMEM((BATCH, HH), jnp.float32),
            pltpu.VMEM((ZCH, HH), jnp.float32),
            pltpu.VMEM_SHARED((ACC_ROWS, HH), jnp.float32),
            pltpu.SemaphoreType.DMA,
            pltpu.SemaphoreType.DMA,
            pltpu.SemaphoreType.DMA,
            pltpu.SemaphoreType.DMA,
        ],
    )
    def k(y_hbm, src_hbm, dst_hbm, out_hbm,
          src_v, dst_v, rows0, rows1, zblk, acc, gs0, gs1, ss0, ss1):
        c = lax.axis_index("c")
        s = lax.axis_index("s")
        y_half = y_hbm.at[c]

        def zero_row(i, carry):
            zblk[i, pl.ds(0, 16)] = jnp.zeros((16,), jnp.float32)
            return carry

        lax.fori_loop(0, ZCH, zero_row, 0)

        def zero_chunk(j, carry):
            pltpu.sync_copy(zblk, acc.at[pl.ds(s * ROWS_PER_TILE + j * ZCH, ZCH)])
            return carry

        lax.fori_loop(0, 4, zero_chunk, 0)
        plsc.subcore_barrier()

        def stage(t, carry):
            pltpu.sync_copy(src_hbm.at[s, pl.ds(t * SB, SB)], src_v)
            pltpu.sync_copy(dst_hbm.at[s, pl.ds(t * SB, SB)], dst_v)

            def body(i, carry2):
                b0 = 2 * i
                b1 = 2 * i + 1
                g0 = pltpu.async_copy(y_half.at[src_v.at[b0]], rows0, gs0)
                g1 = pltpu.async_copy(y_half.at[src_v.at[b1]], rows1, gs1)
                g0.wait()
                s0 = pltpu.async_copy(rows0, acc.at[dst_v.at[b0]], ss0, add=True)
                g1.wait()
                s1 = pltpu.async_copy(rows1, acc.at[dst_v.at[b1]], ss1, add=True)
                s0.wait()
                s1.wait()
                return carry2

            lax.fori_loop(0, SB // 2, body, 0)
            return carry

        lax.fori_loop(0, NSTG, stage, 0)
        plsc.subcore_barrier()

        @pl.when(s == 0)
        def _():
            pltpu.sync_copy(acc.at[pl.ds(0, N)], out_hbm.at[c])

    return k(y2, src_p, dst_p)


_ROW_BLK = 5000


def _tc_pre(x, W_msg, b_msg, W1, b1):
    def body(x_ref, wm_ref, bm_ref, w1_ref, b1_ref, y2_ref, z_ref):
        xb = x_ref[...]
        w1t = w1_ref[:SD, :]
        w1b = w1_ref[SD:, :]
        wt = jnp.dot(wm_ref[...], w1b, preferred_element_type=jnp.float32)
        cvec = jnp.dot(bm_ref[...], w1b, preferred_element_type=jnp.float32)
        yb = jnp.dot(xb, wt, preferred_element_type=jnp.float32) + cvec
        y2_ref[0] = yb[:, :HH]
        y2_ref[1] = yb[:, HH:]
        z_ref[...] = jnp.dot(xb, w1t, preferred_element_type=jnp.float32) + b1_ref[...]

    grid = N // _ROW_BLK
    return pl.pallas_call(
        body,
        grid=(grid,),
        in_specs=[
            pl.BlockSpec((_ROW_BLK, SD), lambda i: (i, 0)),
            pl.BlockSpec((MC, MC), lambda i: (0, 0)),
            pl.BlockSpec((1, MC), lambda i: (0, 0)),
            pl.BlockSpec((SD + MC, H), lambda i: (0, 0)),
            pl.BlockSpec((1, H), lambda i: (0, 0)),
        ],
        out_specs=[
            pl.BlockSpec((NC, _ROW_BLK, HH), lambda i: (0, i, 0)),
            pl.BlockSpec((_ROW_BLK, H), lambda i: (i, 0)),
        ],
        out_shape=[
            jax.ShapeDtypeStruct((NC, N, HH), jnp.float32),
            jax.ShapeDtypeStruct((N, H), jnp.float32),
        ],
    )(x, W_msg, b_msg.reshape(1, MC), W1, b1.reshape(1, H))


def _tc_post(z, s_lo, s_hi, W2, b2, W3, b3):
    def body(z_ref, lo_ref, hi_ref, w2_ref, b2_ref, w3_ref, b3_ref, o_ref):
        s_full = jnp.concatenate([lo_ref[...], hi_ref[...]], axis=1)
        h1 = jnp.maximum(z_ref[...] + s_full, 0.0)
        h2 = jnp.dot(h1, w2_ref[...], preferred_element_type=jnp.float32)
        h2 = jnp.maximum(h2 + b2_ref[...], 0.0)
        o_ref[...] = jnp.dot(h2, w3_ref[...], preferred_element_type=jnp.float32) + b3_ref[...]

    grid = N // _ROW_BLK
    return pl.pallas_call(
        body,
        grid=(grid,),
        in_specs=[
            pl.BlockSpec((_ROW_BLK, H), lambda i: (i, 0)),
            pl.BlockSpec((_ROW_BLK, HH), lambda i: (i, 0)),
            pl.BlockSpec((_ROW_BLK, HH), lambda i: (i, 0)),
            pl.BlockSpec((H, H), lambda i: (0, 0)),
            pl.BlockSpec((1, H), lambda i: (0, 0)),
            pl.BlockSpec((H, SD), lambda i: (0, 0)),
            pl.BlockSpec((1, SD), lambda i: (0, 0)),
        ],
        out_specs=pl.BlockSpec((_ROW_BLK, SD), lambda i: (i, 0)),
        out_shape=jax.ShapeDtypeStruct((N, SD), jnp.float32),
    )(z, s_lo, s_hi, W2, b2.reshape(1, H), W3, b3.reshape(1, SD))


def kernel(x, edge_index, W_msg, b_msg, W1, b1, W2, b2, W3, b3):
    src = edge_index[0]
    dst = edge_index[1]
    pad = EP - E
    src_p = jnp.concatenate([src, jnp.zeros((pad,), jnp.int32)]).reshape(NS, NB, BATCH)
    dst_pad = N + (jnp.arange(pad, dtype=jnp.int32) % (ACC_ROWS - N))
    dst_p = jnp.concatenate([dst, dst_pad]).reshape(NS, NB, BATCH)

    y2, z = _tc_pre(x, W_msg, b_msg, W1, b1)
    s_parts = _sc_segment_sum(y2, src_p, dst_p)
    return _tc_post(z, s_parts[0], s_parts[1], W2, b2, W3, b3)


# speedup vs baseline: 12.2497x; 1.4746x over previous
import functools

import jax
import jax.numpy as jnp
from jax import lax
from jax.experimental import pallas as pl
from jax.experimental.pallas import tpu as pltpu
from jax.experimental.pallas import tpu_sc as plsc

N = 50000
E = 800000
SD = 64
MC = 64
H = 32
HH = 16

NC = 2
NS = 16
BATCH = 128
NROWS = E // BATCH
RMAX = 391
NBUF = 8
SROWS = 16
NST = 24
ACC_ROWS = 50176
ZCH = 196
ROWS_PER_TILE = 16 * ZCH
COPY_LAST = N - 15 * ROWS_PER_TILE


def _sc_segment_sum(y2, ei3):
    mesh = plsc.VectorSubcoreMesh(core_axis_name="c", subcore_axis_name="s")

    @functools.partial(
        pl.kernel,
        out_type=jax.ShapeDtypeStruct((NC, N, HH), jnp.float32),
        mesh=mesh,
        compiler_params=pltpu.CompilerParams(use_tc_tiling_on_sc=False),
        scratch_types=[
            pltpu.VMEM((SROWS, BATCH), jnp.int32),
            pltpu.VMEM((SROWS, BATCH), jnp.int32),
            [pltpu.VMEM((BATCH, HH), jnp.float32)] * NBUF,
            pltpu.VMEM((ZCH, HH), jnp.float32),
            pltpu.VMEM_SHARED((ACC_ROWS, HH), jnp.float32),
            [pltpu.SemaphoreType.DMA] * NBUF,
            [pltpu.SemaphoreType.DMA] * NBUF,
            pltpu.SemaphoreType.DMA,
        ],
    )
    def k(y_hbm, ei_hbm, out_hbm,
          src_v, dst_v, rbufs, zblk, acc, gsems, ssems, misc_sem):
        c = lax.axis_index("c")
        s = lax.axis_index("s")
        y_half = y_hbm.at[c]
        nrows = jnp.where(s < 10, RMAX, RMAX - 1)
        base = s * RMAX - jnp.maximum(s - 10, 0)

        def zero_row(i, carry):
            zblk[i, pl.ds(0, HH)] = jnp.zeros((HH,), jnp.float32)
            return carry

        lax.fori_loop(0, ZCH, zero_row, 0)

        def zero_chunk(j, carry):
            pltpu.sync_copy(zblk, acc.at[pl.ds(s * ROWS_PER_TILE + j * ZCH, ZCH)])
            return carry

        lax.fori_loop(0, 16, zero_chunk, 0)

        plsc.subcore_barrier()

        def stage(t, carry):
            row0 = base + t * SROWS
            pltpu.sync_copy(ei_hbm.at[0, pl.ds(row0, SROWS)], src_v)
            pltpu.sync_copy(ei_hbm.at[1, pl.ds(row0, SROWS)], dst_v)
            for g in range(SROWS // NBUF):
                gh = []
                for j in range(NBUF):
                    gh.append(pltpu.async_copy(
                        y_half.at[src_v.at[g * NBUF + j]], rbufs[j], gsems[j]))
                sh = []
                for j in range(NBUF):
                    gh[j].wait()
                    sh.append(pltpu.async_copy(
                        rbufs[j], acc.at[dst_v.at[g * NBUF + j]], ssems[j],
                        add=True))
                for j in range(NBUF):
                    sh[j].wait()
            return carry

        lax.fori_loop(0, NST, stage, 0)

        tail_sb = base + NST * SROWS
        clamped = jnp.minimum(tail_sb, NROWS - NBUF)
        delta = tail_sb - clamped
        pltpu.sync_copy(ei_hbm.at[0, pl.ds(clamped, NBUF)],
                        src_v.at[pl.ds(0, NBUF)])
        pltpu.sync_copy(ei_hbm.at[1, pl.ds(clamped, NBUF)],
                        dst_v.at[pl.ds(0, NBUF)])

        def tail(t, carry):
            pltpu.async_copy(y_half.at[src_v.at[delta + t]], rbufs[0],
                             gsems[0]).wait()
            pltpu.async_copy(rbufs[0], acc.at[dst_v.at[delta + t]], ssems[0],
                             add=True).wait()
            return carry

        lax.fori_loop(0, nrows - NST * SROWS, tail, 0)
        plsc.subcore_barrier()


        out_c = out_hbm.at[c]

        @pl.when(s < 15)
        def _():
            pltpu.sync_copy(acc.at[pl.ds(s * ROWS_PER_TILE, ROWS_PER_TILE)],
                            out_c.at[pl.ds(s * ROWS_PER_TILE, ROWS_PER_TILE)])

        @pl.when(s == 15)
        def _():
            pltpu.sync_copy(acc.at[pl.ds(15 * ROWS_PER_TILE, COPY_LAST)],
                            out_c.at[pl.ds(15 * ROWS_PER_TILE, COPY_LAST)])

    return k(y2, ei3)


_ROW_BLK = 5000


def _tc_pre(x, W_msg, b_msg, W1, b1):
    def body(x_ref, wm_ref, bm_ref, w1_ref, b1_ref, y2_ref, z_ref):
        xb = x_ref[...]
        w1t = w1_ref[:SD, :]
        w1b = w1_ref[SD:, :]
        wt = jnp.dot(wm_ref[...], w1b, preferred_element_type=jnp.float32)
        cvec = jnp.dot(bm_ref[...], w1b, preferred_element_type=jnp.float32)
        yb = jnp.dot(xb, wt, preferred_element_type=jnp.float32) + cvec
        y2_ref[0] = yb[:, :HH]
        y2_ref[1] = yb[:, HH:]
        z_ref[...] = jnp.dot(xb, w1t, preferred_element_type=jnp.float32) + b1_ref[...]

    grid = N // _ROW_BLK
    return pl.pallas_call(
        body,
        grid=(grid,),
        in_specs=[
            pl.BlockSpec((_ROW_BLK, SD), lambda i: (i, 0)),
            pl.BlockSpec((MC, MC), lambda i: (0, 0)),
            pl.BlockSpec((1, MC), lambda i: (0, 0)),
            pl.BlockSpec((SD + MC, H), lambda i: (0, 0)),
            pl.BlockSpec((1, H), lambda i: (0, 0)),
        ],
        out_specs=[
            pl.BlockSpec((NC, _ROW_BLK, HH), lambda i: (0, i, 0)),
            pl.BlockSpec((_ROW_BLK, H), lambda i: (i, 0)),
        ],
        out_shape=[
            jax.ShapeDtypeStruct((NC, N, HH), jnp.float32),
            jax.ShapeDtypeStruct((N, H), jnp.float32),
        ],
    )(x, W_msg, b_msg.reshape(1, MC), W1, b1.reshape(1, H))


def _tc_post(z, s2, W2, b2, W3, b3):
    def body(z_ref, s2_ref, w2_ref, b2_ref, w3_ref, b3_ref, o_ref):
        s_full = jnp.concatenate([s2_ref[0], s2_ref[1]], axis=1)
        h1 = jnp.maximum(z_ref[...] + s_full, 0.0)
        h2 = jnp.dot(h1, w2_ref[...], preferred_element_type=jnp.float32)
        h2 = jnp.maximum(h2 + b2_ref[...], 0.0)
        o_ref[...] = jnp.dot(h2, w3_ref[...], preferred_element_type=jnp.float32) + b3_ref[...]

    grid = N // _ROW_BLK
    return pl.pallas_call(
        body,
        grid=(grid,),
        in_specs=[
            pl.BlockSpec((_ROW_BLK, H), lambda i: (i, 0)),
            pl.BlockSpec((NC, _ROW_BLK, HH), lambda i: (0, i, 0)),
            pl.BlockSpec((H, H), lambda i: (0, 0)),
            pl.BlockSpec((1, H), lambda i: (0, 0)),
            pl.BlockSpec((H, SD), lambda i: (0, 0)),
            pl.BlockSpec((1, SD), lambda i: (0, 0)),
        ],
        out_specs=pl.BlockSpec((_ROW_BLK, SD), lambda i: (i, 0)),
        out_shape=jax.ShapeDtypeStruct((N, SD), jnp.float32),
    )(z, s2, W2, b2.reshape(1, H), W3, b3.reshape(1, SD))


def kernel(x, edge_index, W_msg, b_msg, W1, b1, W2, b2, W3, b3):
    ei3 = edge_index.reshape(2, NROWS, BATCH)
    y2, z = _tc_pre(x, W_msg, b_msg, W1, b1)
    s2 = _sc_segment_sum(y2, ei3)
    return _tc_post(z, s2, W2, b2, W3, b3)


# speedup vs baseline: 14.3052x; 1.1678x over previous
import functools

import jax
import jax.numpy as jnp
from jax import lax
from jax.experimental import pallas as pl
from jax.experimental.pallas import tpu as pltpu
from jax.experimental.pallas import tpu_sc as plsc

N = 50000
E = 800000
SD = 64
MC = 64
H = 32
HH = 16

NC = 2
NS = 16
BATCH = 128
NROWS = E // BATCH
RMAX = 391
NBUF = 10
SROWS = 10
NST = 39
NIB = 3
ACC_ROWS = 50176
ROWS_PER_TILE = 3136
COPY_LAST = N - 15 * ROWS_PER_TILE


def _sc_segment_sum(y2, ei3, zeros):
    mesh = plsc.VectorSubcoreMesh(core_axis_name="c", subcore_axis_name="s")

    @functools.partial(
        pl.kernel,
        out_type=jax.ShapeDtypeStruct((NC, N, 128), jnp.float32),
        mesh=mesh,
        compiler_params=pltpu.CompilerParams(use_tc_tiling_on_sc=False),
        scratch_types=[
            [pltpu.VMEM((SROWS, BATCH), jnp.int32)] * NIB,
            [pltpu.VMEM((SROWS, BATCH), jnp.int32)] * NIB,
            [pltpu.VMEM((BATCH, HH), jnp.float32)] * NBUF,
            pltpu.VMEM_SHARED((ACC_ROWS, HH), jnp.float32),
            [pltpu.SemaphoreType.DMA] * NBUF,
            [pltpu.SemaphoreType.DMA] * NBUF,
            [pltpu.SemaphoreType.DMA] * NIB,
            [pltpu.SemaphoreType.DMA] * NIB,
        ],
    )
    def k(y_hbm, ei_hbm, zeros_hbm, out_hbm,
          src_vs, dst_vs, rbufs, acc, gsems, ssems, sisems, disems):
        c = lax.axis_index("c")
        s = lax.axis_index("s")
        y_half = y_hbm.at[c]
        base = s * RMAX - jnp.maximum(s - 10, 0)

        pltpu.sync_copy(zeros_hbm,
                        acc.at[pl.ds(s * ROWS_PER_TILE, ROWS_PER_TILE)])
        plsc.subcore_barrier()

        def issue_idx(k_stage, m):
            row0 = base + k_stage * SROWS
            pltpu.async_copy(ei_hbm.at[0, pl.ds(row0, SROWS)], src_vs[m],
                             sisems[m])
            pltpu.async_copy(ei_hbm.at[1, pl.ds(row0, SROWS)], dst_vs[m],
                             disems[m])

        def wait_idx(k_stage, m):
            row0 = base + k_stage * SROWS
            pltpu.make_async_copy(ei_hbm.at[0, pl.ds(row0, SROWS)], src_vs[m],
                                  sisems[m]).wait()
            pltpu.make_async_copy(ei_hbm.at[1, pl.ds(row0, SROWS)], dst_vs[m],
                                  disems[m]).wait()

        for m in range(NIB):
            issue_idx(m, m)

        def outer(i, carry):
            for m in range(NIB):
                k_stage = NIB * i + m
                wait_idx(k_stage, m)
                gh = []
                for j in range(NBUF):
                    gh.append(pltpu.async_copy(
                        y_half.at[src_vs[m].at[j]], rbufs[j], gsems[j]))
                sh = []
                for j in range(NBUF):
                    gh[j].wait()
                    sh.append(pltpu.async_copy(
                        rbufs[j], acc.at[dst_vs[m].at[j]], ssems[j],
                        add=True))
                for j in range(NBUF):
                    sh[j].wait()

                @pl.when(k_stage + NIB < NST)
                def _():
                    issue_idx(k_stage + NIB, m)
            return carry

        lax.fori_loop(0, NST // NIB, outer, 0)

        @pl.when(s < 10)
        def _():
            pltpu.sync_copy(ei_hbm.at[0, pl.ds(base + NST * SROWS, 1)],
                            src_vs[0].at[pl.ds(0, 1)])
            pltpu.sync_copy(ei_hbm.at[1, pl.ds(base + NST * SROWS, 1)],
                            dst_vs[0].at[pl.ds(0, 1)])
            pltpu.async_copy(y_half.at[src_vs[0].at[0]], rbufs[0],
                             gsems[0]).wait()
            pltpu.async_copy(rbufs[0], acc.at[dst_vs[0].at[0]], ssems[0],
                             add=True).wait()

        plsc.subcore_barrier()

        out_c = out_hbm.at[c]

        @pl.when(s < 15)
        def _():
            pltpu.sync_copy(
                acc.at[pl.ds(s * ROWS_PER_TILE, ROWS_PER_TILE)],
                out_c.at[pl.ds(s * ROWS_PER_TILE, ROWS_PER_TILE), pl.ds(0, HH)])

        @pl.when(s == 15)
        def _():
            pltpu.sync_copy(
                acc.at[pl.ds(15 * ROWS_PER_TILE, COPY_LAST)],
                out_c.at[pl.ds(15 * ROWS_PER_TILE, COPY_LAST), pl.ds(0, HH)])

    return k(y2, ei3, zeros)


_ROW_BLK = 5000


def _tc_pre(x, W_msg, b_msg, W1):
    def body(x_ref, wm_ref, bm_ref, w1_ref, y2_ref):
        xb = x_ref[...]
        w1b = w1_ref[SD:, :]
        wt = jnp.dot(wm_ref[...], w1b, preferred_element_type=jnp.float32)
        cvec = jnp.dot(bm_ref[...], w1b, preferred_element_type=jnp.float32)
        yb = jnp.dot(xb, wt, preferred_element_type=jnp.float32) + cvec
        y2_ref[0] = yb[:, :HH]
        y2_ref[1] = yb[:, HH:]

    grid = N // _ROW_BLK
    return pl.pallas_call(
        body,
        grid=(grid,),
        in_specs=[
            pl.BlockSpec((_ROW_BLK, SD), lambda i: (i, 0)),
            pl.BlockSpec((MC, MC), lambda i: (0, 0)),
            pl.BlockSpec((1, MC), lambda i: (0, 0)),
            pl.BlockSpec((SD + MC, H), lambda i: (0, 0)),
        ],
        out_specs=pl.BlockSpec((NC, _ROW_BLK, HH), lambda i: (0, i, 0)),
        out_shape=jax.ShapeDtypeStruct((NC, N, HH), jnp.float32),
    )(x, W_msg, b_msg.reshape(1, MC), W1)


def _tc_post(x, s2p, W1, b1, W2, b2, W3, b3):
    def body(x_ref, s2_ref, w1_ref, b1_ref, w2_ref, b2_ref, w3_ref, b3_ref,
             o_ref):
        xb = x_ref[...]
        z = jnp.dot(xb, w1_ref[:SD, :], preferred_element_type=jnp.float32)
        s_full = jnp.concatenate(
            [s2_ref[0][:, :HH], s2_ref[1][:, :HH]], axis=1)
        h1 = jnp.maximum(z + b1_ref[...] + s_full, 0.0)
        h2 = jnp.dot(h1, w2_ref[...], preferred_element_type=jnp.float32)
        h2 = jnp.maximum(h2 + b2_ref[...], 0.0)
        o_ref[...] = jnp.dot(h2, w3_ref[...], preferred_element_type=jnp.float32) + b3_ref[...]

    grid = N // _ROW_BLK
    return pl.pallas_call(
        body,
        grid=(grid,),
        in_specs=[
            pl.BlockSpec((_ROW_BLK, SD), lambda i: (i, 0)),
            pl.BlockSpec((NC, _ROW_BLK, 128), lambda i: (0, i, 0)),
            pl.BlockSpec((SD + MC, H), lambda i: (0, 0)),
            pl.BlockSpec((1, H), lambda i: (0, 0)),
            pl.BlockSpec((H, H), lambda i: (0, 0)),
            pl.BlockSpec((1, H), lambda i: (0, 0)),
            pl.BlockSpec((H, SD), lambda i: (0, 0)),
            pl.BlockSpec((1, SD), lambda i: (0, 0)),
        ],
        out_specs=pl.BlockSpec((_ROW_BLK, SD), lambda i: (i, 0)),
        out_shape=jax.ShapeDtypeStruct((N, SD), jnp.float32),
    )(x, s2p, W1, b1.reshape(1, H), W2, b2.reshape(1, H), W3,
      b3.reshape(1, SD))


def kernel(x, edge_index, W_msg, b_msg, W1, b1, W2, b2, W3, b3):
    ei3 = edge_index.reshape(2, NROWS, BATCH)
    y2 = _tc_pre(x, W_msg, b_msg, W1)
    zeros = jnp.zeros((ROWS_PER_TILE, HH), jnp.float32)
    s2p = _sc_segment_sum(y2, ei3, zeros)
    return _tc_post(x, s2p, W1, b1, W2, b2, W3, b3)


# speedup vs baseline: 14.4312x; 1.0088x over previous
import functools

import jax
import jax.numpy as jnp
from jax import lax
from jax.experimental import pallas as pl
from jax.experimental.pallas import tpu as pltpu
from jax.experimental.pallas import tpu_sc as plsc

N = 50000
E = 800000
SD = 64
MC = 64
H = 32
HH = 16

NC = 2
NS = 16
BATCH = 128
NROWS = E // BATCH
RMAX = 391
NBUF = 10
SROWS = 10
NST = 39
NIB = 3
ACC_ROWS = 50176
ZCH = 98
ROWS_PER_TILE = 32 * ZCH
COPY_LAST = N - 15 * ROWS_PER_TILE


def _sc_segment_sum(y2, ei3):
    mesh = plsc.VectorSubcoreMesh(core_axis_name="c", subcore_axis_name="s")

    @functools.partial(
        pl.kernel,
        out_type=jax.ShapeDtypeStruct((NC, N, 128), jnp.float32),
        mesh=mesh,
        compiler_params=pltpu.CompilerParams(use_tc_tiling_on_sc=False),
        scratch_types=[
            [pltpu.VMEM((SROWS, BATCH), jnp.int32)] * NIB,
            [pltpu.VMEM((SROWS, BATCH), jnp.int32)] * NIB,
            [pltpu.VMEM((BATCH, HH), jnp.float32)] * NBUF,
            pltpu.VMEM((ZCH, HH), jnp.float32),
            pltpu.VMEM_SHARED((ACC_ROWS, HH), jnp.float32),
            [pltpu.SemaphoreType.DMA] * NBUF,
            [pltpu.SemaphoreType.DMA] * NBUF,
            [pltpu.SemaphoreType.DMA] * NIB,
            [pltpu.SemaphoreType.DMA] * NIB,
        ],
    )
    def k(y_hbm, ei_hbm, out_hbm,
          src_vs, dst_vs, rbufs, zblk, acc, gsems, ssems, sisems, disems):
        c = lax.axis_index("c")
        s = lax.axis_index("s")
        y_half = y_hbm.at[c]
        base = s * RMAX - jnp.maximum(s - 10, 0)

        def zero_row(i, carry):
            zblk[i, pl.ds(0, HH)] = jnp.zeros((HH,), jnp.float32)
            return carry

        lax.fori_loop(0, ZCH, zero_row, 0)

        def zero_chunk(j, carry):
            pltpu.sync_copy(zblk, acc.at[pl.ds(s * ROWS_PER_TILE + j * ZCH, ZCH)])
            return carry

        lax.fori_loop(0, ROWS_PER_TILE // ZCH, zero_chunk, 0)
        plsc.subcore_barrier()

        def issue_idx(k_stage, m):
            row0 = base + k_stage * SROWS
            pltpu.async_copy(ei_hbm.at[0, pl.ds(row0, SROWS)], src_vs[m],
                             sisems[m])
            pltpu.async_copy(ei_hbm.at[1, pl.ds(row0, SROWS)], dst_vs[m],
                             disems[m])

        def wait_idx(k_stage, m):
            row0 = base + k_stage * SROWS
            pltpu.make_async_copy(ei_hbm.at[0, pl.ds(row0, SROWS)], src_vs[m],
                                  sisems[m]).wait()
            pltpu.make_async_copy(ei_hbm.at[1, pl.ds(row0, SROWS)], dst_vs[m],
                                  disems[m]).wait()

        for m in range(NIB):
            issue_idx(m, m)

        def outer(i, carry):
            for m in range(NIB):
                k_stage = NIB * i + m
                wait_idx(k_stage, m)
                gh = []
                for j in range(NBUF):
                    gh.append(pltpu.async_copy(
                        y_half.at[src_vs[m].at[j]], rbufs[j], gsems[j]))
                sh = []
                for j in range(NBUF):
                    gh[j].wait()
                    sh.append(pltpu.async_copy(
                        rbufs[j], acc.at[dst_vs[m].at[j]], ssems[j],
                        add=True))
                for j in range(NBUF):
                    sh[j].wait()

                @pl.when(k_stage + NIB < NST)
                def _():
                    issue_idx(k_stage + NIB, m)
            return carry

        lax.fori_loop(0, NST // NIB, outer, 0)

        @pl.when(s < 10)
        def _():
            pltpu.sync_copy(ei_hbm.at[0, pl.ds(base + NST * SROWS, 1)],
                            src_vs[0].at[pl.ds(0, 1)])
            pltpu.sync_copy(ei_hbm.at[1, pl.ds(base + NST * SROWS, 1)],
                            dst_vs[0].at[pl.ds(0, 1)])
            pltpu.async_copy(y_half.at[src_vs[0].at[0]], rbufs[0],
                             gsems[0]).wait()
            pltpu.async_copy(rbufs[0], acc.at[dst_vs[0].at[0]], ssems[0],
                             add=True).wait()

        plsc.subcore_barrier()

        out_c = out_hbm.at[c]

        @pl.when(s < 15)
        def _():
            pltpu.sync_copy(
                acc.at[pl.ds(s * ROWS_PER_TILE, ROWS_PER_TILE)],
                out_c.at[pl.ds(s * ROWS_PER_TILE, ROWS_PER_TILE), pl.ds(0, HH)])

        @pl.when(s == 15)
        def _():
            pltpu.sync_copy(
                acc.at[pl.ds(15 * ROWS_PER_TILE, COPY_LAST)],
                out_c.at[pl.ds(15 * ROWS_PER_TILE, COPY_LAST), pl.ds(0, HH)])

    return k(y2, ei3)


_ROW_BLK = 5000


def _tc_pre(x, W_msg, b_msg, W1):
    def body(x_ref, wm_ref, bm_ref, w1_ref, y2_ref):
        xb = x_ref[...]
        w1b = w1_ref[SD:, :]
        wt = jnp.dot(wm_ref[...], w1b, preferred_element_type=jnp.float32)
        cvec = jnp.dot(bm_ref[...], w1b, preferred_element_type=jnp.float32)
        yb = jnp.dot(xb, wt, preferred_element_type=jnp.float32) + cvec
        y2_ref[0] = yb[:, :HH]
        y2_ref[1] = yb[:, HH:]

    grid = N // _ROW_BLK
    return pl.pallas_call(
        body,
        grid=(grid,),
        in_specs=[
            pl.BlockSpec((_ROW_BLK, SD), lambda i: (i, 0)),
            pl.BlockSpec((MC, MC), lambda i: (0, 0)),
            pl.BlockSpec((1, MC), lambda i: (0, 0)),
            pl.BlockSpec((SD + MC, H), lambda i: (0, 0)),
        ],
        out_specs=pl.BlockSpec((NC, _ROW_BLK, HH), lambda i: (0, i, 0)),
        out_shape=jax.ShapeDtypeStruct((NC, N, HH), jnp.float32),
    )(x, W_msg, b_msg.reshape(1, MC), W1)


def _tc_post(x, s2p, W1, b1, W2, b2, W3, b3):
    def body(x_ref, s2_ref, w1_ref, b1_ref, w2_ref, b2_ref, w3_ref, b3_ref,
             o_ref):
        xb = x_ref[...]
        z = jnp.dot(xb, w1_ref[:SD, :], preferred_element_type=jnp.float32)
        s_full = jnp.concatenate(
            [s2_ref[0][:, :HH], s2_ref[1][:, :HH]], axis=1)
        h1 = jnp.maximum(z + b1_ref[...] + s_full, 0.0)
        h2 = jnp.dot(h1, w2_ref[...], preferred_element_type=jnp.float32)
        h2 = jnp.maximum(h2 + b2_ref[...], 0.0)
        o_ref[...] = jnp.dot(h2, w3_ref[...], preferred_element_type=jnp.float32) + b3_ref[...]

    grid = N // _ROW_BLK
    return pl.pallas_call(
        body,
        grid=(grid,),
        in_specs=[
            pl.BlockSpec((_ROW_BLK, SD), lambda i: (i, 0)),
            pl.BlockSpec((NC, _ROW_BLK, 128), lambda i: (0, i, 0)),
            pl.BlockSpec((SD + MC, H), lambda i: (0, 0)),
            pl.BlockSpec((1, H), lambda i: (0, 0)),
            pl.BlockSpec((H, H), lambda i: (0, 0)),
            pl.BlockSpec((1, H), lambda i: (0, 0)),
            pl.BlockSpec((H, SD), lambda i: (0, 0)),
            pl.BlockSpec((1, SD), lambda i: (0, 0)),
        ],
        out_specs=pl.BlockSpec((_ROW_BLK, SD), lambda i: (i, 0)),
        out_shape=jax.ShapeDtypeStruct((N, SD), jnp.float32),
    )(x, s2p, W1, b1.reshape(1, H), W2, b2.reshape(1, H), W3,
      b3.reshape(1, SD))


def kernel(x, edge_index, W_msg, b_msg, W1, b1, W2, b2, W3, b3):
    ei3 = edge_index.reshape(2, NROWS, BATCH)
    y2 = _tc_pre(x, W_msg, b_msg, W1)
    s2p = _sc_segment_sum(y2, ei3)
    return _tc_post(x, s2p, W1, b1, W2, b2, W3, b3)
